# Initial kernel scaffold; baseline (speedup 1.0000x reference)
import jax
import jax.numpy as jnp
from jax.experimental import pallas as pl


def kernel(pointcloud, params, numpoints):
    raise NotImplementedError("write your pallas kernel here")


# speedup vs baseline: 5.1983x; 5.1983x over previous
import functools

import jax
import jax.numpy as jnp
from jax.experimental import pallas as pl

_EPS = 1e-5
_NPTS = (1024, 256, 64)
_F32 = jnp.float32


def _fps_body(start_ref, x_ref, y_ref, z_ref, fx_ref, fy_ref, fz_ref,
              fi_ref, *, npoint):
    x = x_ref[...]
    y = y_ref[...]
    z = z_ref[...]
    b, n = x.shape
    iota = jax.lax.broadcasted_iota(jnp.int32, (b, n), 1).astype(_F32)
    iop = jax.lax.broadcasted_iota(jnp.int32, (b, npoint), 1)
    far0 = start_ref[...].astype(_F32)

    def body(i, st):
        dist, far, fx, fy, fz, fi = st
        onehot = iota == far
        cx = jnp.sum(jnp.where(onehot, x, 0.0), axis=1, keepdims=True)
        cy = jnp.sum(jnp.where(onehot, y, 0.0), axis=1, keepdims=True)
        cz = jnp.sum(jnp.where(onehot, z, 0.0), axis=1, keepdims=True)
        sel = iop == i
        fx = jnp.where(sel, cx, fx)
        fy = jnp.where(sel, cy, fy)
        fz = jnp.where(sel, cz, fz)
        fi = jnp.where(sel, far.astype(jnp.int32), fi)
        d = (x - cx) ** 2 + (y - cy) ** 2 + (z - cz) ** 2
        dist = jnp.minimum(dist, d)
        m = jnp.max(dist, axis=1, keepdims=True)
        far = jnp.min(jnp.where(dist == m, iota, float(n)), axis=1,
                      keepdims=True)
        return dist, far, fx, fy, fz, fi

    init = (jnp.full((b, n), 1e10, _F32), far0,
            jnp.zeros((b, npoint), _F32), jnp.zeros((b, npoint), _F32),
            jnp.zeros((b, npoint), _F32), jnp.zeros((b, npoint), jnp.int32))
    _, _, fx, fy, fz, fi = jax.lax.fori_loop(0, npoint, body, init)
    fx_ref[...] = fx
    fy_ref[...] = fy
    fz_ref[...] = fz
    fi_ref[...] = fi


def _fps(x, y, z, start, npoint):
    b, n = x.shape
    sd = jax.ShapeDtypeStruct
    return pl.pallas_call(
        functools.partial(_fps_body, npoint=npoint),
        out_shape=[sd((b, npoint), _F32), sd((b, npoint), _F32),
                   sd((b, npoint), _F32), sd((b, npoint), jnp.int32)],
    )(start, x, y, z)


def _sqdist(a, pts3, xp_ref, yp_ref, zp_ref):
    bx, by, bz = xp_ref[0], yp_ref[0], zp_ref[0]
    sa = jnp.sum(a * a, axis=1, keepdims=True)
    sb = bx * bx + by * by + bz * bz
    ab = jax.lax.dot_general(a, pts3, (((1,), (1,)), ((), ())),
                             preferred_element_type=_F32)
    return sa + sb - 2.0 * ab


def _ball_body(fxyz_ref, xp_ref, yp_ref, zp_ref, pts_ref, out_ref, *, k, r2):
    a = fxyz_ref[0]
    d = _sqdist(a, pts_ref[0], xp_ref, yp_ref, zp_ref)
    sn, n = d.shape
    mask = d <= r2
    mf = mask.astype(_F32)
    c = mf
    s = 1
    while s < n:
        c = c + jnp.concatenate(
            [jnp.zeros((sn, s), _F32), c[:, :n - s]], axis=1)
        s *= 2
    rank = c - mf
    cnt = jnp.sum(mf, axis=1, keepdims=True)
    pts = pts_ref[0]
    last = jnp.broadcast_to(pts[n - 1:n, :], (sn, 3))

    def body(j, g0):
        jf = j.astype(_F32)
        oh = jnp.where(mask & (rank == jf), 1.0, 0.0)
        g = jnp.dot(oh, pts, precision=jax.lax.Precision.HIGHEST,
                    preferred_element_type=_F32)
        g0 = jnp.where(j == 0, g, g0)
        g0 = jnp.where(cnt > 0, g0, last)
        gk = jnp.where(jf < cnt, g, g0)
        out_ref[0, pl.ds(j, 1), :, :] = (gk - a)[None]
        return g0

    jax.lax.fori_loop(0, k, body, jnp.zeros((sn, 3), _F32))


def _ball_group(fxyz, xp, yp, zp, pts, k, r2, sn):
    b, s, _ = fxyz.shape
    n = pts.shape[1]
    return pl.pallas_call(
        functools.partial(_ball_body, k=k, r2=r2),
        grid=(b, s // sn),
        in_specs=[
            pl.BlockSpec((1, sn, 3), lambda i, j: (i, j, 0)),
            pl.BlockSpec((1, 1, n), lambda i, j: (i, 0, 0)),
            pl.BlockSpec((1, 1, n), lambda i, j: (i, 0, 0)),
            pl.BlockSpec((1, 1, n), lambda i, j: (i, 0, 0)),
            pl.BlockSpec((1, n, 3), lambda i, j: (i, 0, 0)),
        ],
        out_specs=pl.BlockSpec((1, k, sn, 3), lambda i, j: (i, 0, j, 0)),
        out_shape=jax.ShapeDtypeStruct((b, k, s, 3), _F32),
    )(fxyz, xp, yp, zp, pts)


def _knn_body(fxyz_ref, xp_ref, yp_ref, zp_ref, pts_ref, out_ref, *, k):
    a = fxyz_ref[0]
    pts = pts_ref[0]
    d = _sqdist(a, pts[:, 0:3], xp_ref, yp_ref, zp_ref)
    s, n = d.shape
    iota = jax.lax.broadcasted_iota(jnp.int32, (s, n), 1).astype(_F32)

    def body(j, d):
        m = jnp.min(d, axis=1, keepdims=True)
        jstar = jnp.min(jnp.where(d == m, iota, float(n)), axis=1,
                        keepdims=True)
        oh = iota == jstar
        g = jnp.dot(oh.astype(_F32), pts, precision=jax.lax.Precision.HIGHEST,
                    preferred_element_type=_F32)
        row = jnp.concatenate([g[:, :3] - a, g[:, 3:]], axis=1)
        out_ref[0, pl.ds(j, 1), :, :] = row[None]
        return jnp.where(oh, 1e30, d)

    jax.lax.fori_loop(0, k, body, d)


def _knn_group(fxyz, xp, yp, zp, pts, k):
    b, s, _ = fxyz.shape
    n, c = pts.shape[1], pts.shape[2]
    return pl.pallas_call(
        functools.partial(_knn_body, k=k),
        grid=(b,),
        in_specs=[
            pl.BlockSpec((1, s, 3), lambda i: (i, 0, 0)),
            pl.BlockSpec((1, 1, n), lambda i: (i, 0, 0)),
            pl.BlockSpec((1, 1, n), lambda i: (i, 0, 0)),
            pl.BlockSpec((1, 1, n), lambda i: (i, 0, 0)),
            pl.BlockSpec((1, n, c), lambda i: (i, 0, 0)),
        ],
        out_specs=pl.BlockSpec((1, k, s, c), lambda i: (i, 0, 0, 0)),
        out_shape=jax.ShapeDtypeStruct((b, k, s, c), _F32),
    )(fxyz, xp, yp, zp, pts)


def _mm_body(x_ref, a_ref, c_ref, w_ref, b_ref, y_ref, s1_ref, s2_ref, *,
             act):
    x = x_ref[...]
    if act:
        x = jnp.maximum(x * a_ref[...] + c_ref[...], 0.0)
    y = jnp.dot(x, w_ref[...], preferred_element_type=_F32) + b_ref[...]
    y_ref[...] = y

    @pl.when(pl.program_id(0) == 0)
    def _():
        s1_ref[...] = jnp.zeros_like(s1_ref)
        s2_ref[...] = jnp.zeros_like(s2_ref)

    s1_ref[...] += jnp.sum(y, axis=0, keepdims=True)
    s2_ref[...] += jnp.sum(y * y, axis=0, keepdims=True)


def _mm_stats(x, a, c, w, bias, act, bm=2048):
    r, ci = x.shape
    co = w.shape[1]
    bm = min(bm, r)
    sd = jax.ShapeDtypeStruct
    return pl.pallas_call(
        functools.partial(_mm_body, act=act),
        grid=(r // bm,),
        in_specs=[
            pl.BlockSpec((bm, ci), lambda i: (i, 0)),
            pl.BlockSpec((1, ci), lambda i: (0, 0)),
            pl.BlockSpec((1, ci), lambda i: (0, 0)),
            pl.BlockSpec((ci, co), lambda i: (0, 0)),
            pl.BlockSpec((1, co), lambda i: (0, 0)),
        ],
        out_specs=[
            pl.BlockSpec((bm, co), lambda i: (i, 0)),
            pl.BlockSpec((1, co), lambda i: (0, 0)),
            pl.BlockSpec((1, co), lambda i: (0, 0)),
        ],
        out_shape=[sd((r, co), _F32), sd((1, co), _F32), sd((1, co), _F32)],
    )(x, a, c, w, bias)


def _pool_body(y_ref, a_ref, c_ref, o_ref):
    t = jnp.maximum(y_ref[0] * a_ref[...][None] + c_ref[...][None], 0.0)
    o_ref[0] = jnp.max(t, axis=0)


def _pool(y, a, c):
    b, k, s, ch = y.shape
    return pl.pallas_call(
        _pool_body,
        grid=(b,),
        in_specs=[
            pl.BlockSpec((1, k, s, ch), lambda i: (i, 0, 0, 0)),
            pl.BlockSpec((1, ch), lambda i: (0, 0)),
            pl.BlockSpec((1, ch), lambda i: (0, 0)),
        ],
        out_specs=pl.BlockSpec((1, s, ch), lambda i: (i, 0, 0)),
        out_shape=jax.ShapeDtypeStruct((b, s, ch), _F32),
    )(y, a, c)


def _bn_relu_body(y_ref, a_ref, c_ref, o_ref):
    o_ref[...] = jnp.maximum(y_ref[...] * a_ref[...] + c_ref[...], 0.0)


def _bn_relu(y, a, c, bm=2048):
    r, ch = y.shape
    bm = min(bm, r)
    return pl.pallas_call(
        _bn_relu_body,
        grid=(r // bm,),
        in_specs=[
            pl.BlockSpec((bm, ch), lambda i: (i, 0)),
            pl.BlockSpec((1, ch), lambda i: (0, 0)),
            pl.BlockSpec((1, ch), lambda i: (0, 0)),
        ],
        out_specs=pl.BlockSpec((bm, ch), lambda i: (i, 0)),
        out_shape=jax.ShapeDtypeStruct((r, ch), _F32),
    )(y, a, c)


def _interp_body(xyz1_ref, xyz2_ref, x2_ref, y2_ref, z2_ref, f2_ref, o_ref):
    a = xyz1_ref[0]
    d = _sqdist(a, xyz2_ref[0], x2_ref, y2_ref, z2_ref)
    rn, n2 = d.shape
    iota = jax.lax.broadcasted_iota(jnp.int32, (rn, n2), 1).astype(_F32)
    wm = jnp.zeros((rn, n2), _F32)
    wsum = jnp.zeros((rn, 1), _F32)
    for _ in range(3):
        m = jnp.min(d, axis=1, keepdims=True)
        jstar = jnp.min(jnp.where(d == m, iota, float(n2)), axis=1,
                        keepdims=True)
        oh = iota == jstar
        w = 1.0 / jnp.maximum(m, 1e-10)
        wm = wm + jnp.where(oh, w, 0.0)
        wsum = wsum + w
        d = jnp.where(oh, 1e30, d)
    wm = wm / wsum
    o_ref[0] = jnp.dot(wm, f2_ref[0], precision=jax.lax.Precision.HIGHEST,
                       preferred_element_type=_F32)


def _interp(xyz1, xyz2, x2, y2, z2, f2, rn):
    b, n1, _ = xyz1.shape
    n2, c2 = f2.shape[1], f2.shape[2]
    rn = min(rn, n1)
    return pl.pallas_call(
        _interp_body,
        grid=(b, n1 // rn),
        in_specs=[
            pl.BlockSpec((1, rn, 3), lambda i, j: (i, j, 0)),
            pl.BlockSpec((1, n2, 3), lambda i, j: (i, 0, 0)),
            pl.BlockSpec((1, 1, n2), lambda i, j: (i, 0, 0)),
            pl.BlockSpec((1, 1, n2), lambda i, j: (i, 0, 0)),
            pl.BlockSpec((1, 1, n2), lambda i, j: (i, 0, 0)),
            pl.BlockSpec((1, n2, c2), lambda i, j: (i, 0, 0)),
        ],
        out_specs=pl.BlockSpec((1, rn, c2), lambda i, j: (i, j, 0)),
        out_shape=jax.ShapeDtypeStruct((b, n1, c2), _F32),
    )(xyz1, xyz2, x2, y2, z2, f2)


def _final_body(x_ref, w_ref, b_ref, o_ref):
    y = jnp.dot(x_ref[0], w_ref[...], preferred_element_type=_F32)
    o_ref[0] = (y + b_ref[...]).T


def _final_mm(x, w, bias, bn=512):
    b, n, ci = x.shape
    co = w.shape[1]
    return pl.pallas_call(
        _final_body,
        grid=(b, n // bn),
        in_specs=[
            pl.BlockSpec((1, bn, ci), lambda i, j: (i, j, 0)),
            pl.BlockSpec((ci, co), lambda i, j: (0, 0)),
            pl.BlockSpec((1, co), lambda i, j: (0, 0)),
        ],
        out_specs=pl.BlockSpec((1, co, bn), lambda i, j: (i, 0, j)),
        out_shape=jax.ShapeDtypeStruct((b, co, n), _F32),
    )(x, w, bias)


def _bn_affine(s1, s2, r, gamma, beta):
    mean = s1 / r
    var = s2 / r - mean * mean
    a = gamma[None, :] * jax.lax.rsqrt(var + _EPS)
    c = beta[None, :] - mean * a
    return a, c


def _mlp_rows(x, layers, last_act):
    r = x.shape[0]
    a = jnp.zeros((1, x.shape[1]), _F32)
    c = jnp.zeros((1, x.shape[1]), _F32)
    act = False
    for (w, bias, gamma, beta) in layers:
        x, s1, s2 = _mm_stats(x, a, c, w, bias[None, :], act)
        a, c = _bn_affine(s1, s2, float(r), gamma, beta)
        act = True
    return x, a, c


def kernel(pointcloud, params, numpoints):
    b, n, _ = pointcloud.shape
    xyz = pointcloud[..., 0:3]
    xp = [xyz[:, :, 0]]
    yp = [xyz[:, :, 1]]
    zp = [xyz[:, :, 2]]
    xyz3 = [xyz]
    feats = [None]

    sa_cfg = [(0.3, 32), (None, 48), (None, 48)]
    for lvl in range(3):
        radius, k = sa_cfg[lvl]
        npoint = _NPTS[lvl]
        start = (jnp.asarray(numpoints[lvl]).astype(jnp.int32)
                 - jnp.int32(npoint))
        start_arr = jnp.full((b, 1), 1, jnp.int32) * start
        fx, fy, fz, _ = _fps(xp[lvl], yp[lvl], zp[lvl], start_arr, npoint)
        fxyz = jnp.stack([fx, fy, fz], axis=-1)
        if radius is not None:
            pts = xyz3[lvl]
            grouped = _ball_group(fxyz, xp[lvl][:, None, :],
                                  yp[lvl][:, None, :], zp[lvl][:, None, :],
                                  pts, k, radius * radius, sn=512)
        else:
            pts = jnp.concatenate([xyz3[lvl], feats[lvl]], axis=-1)
            grouped = _knn_group(fxyz, xp[lvl][:, None, :],
                                 yp[lvl][:, None, :], zp[lvl][:, None, :],
                                 pts, k)
        cin = grouped.shape[-1]
        rows = grouped.reshape(b * k * npoint, cin)
        y, a, c = _mlp_rows(rows, params['sa'][lvl], last_act=False)
        co = y.shape[1]
        feat = _pool(y.reshape(b, k, npoint, co), a, c)
        xp.append(fx)
        yp.append(fy)
        zp.append(fz)
        xyz3.append(fxyz)
        feats.append(feat)

    for lvl in (2, 1, 0):
        n1 = xyz3[lvl].shape[1]
        interp = _interp(xyz3[lvl], xyz3[lvl + 1], xp[lvl + 1][:, None, :],
                         yp[lvl + 1][:, None, :], zp[lvl + 1][:, None, :],
                         feats[lvl + 1], rn=1024)
        if feats[lvl] is not None:
            x0 = jnp.concatenate([interp, feats[lvl]], axis=-1)
        else:
            x0 = interp
        rows = x0.reshape(b * n1, x0.shape[-1])
        y, a, c = _mlp_rows(rows, params['fp'][lvl], last_act=False)
        feats[lvl] = _bn_relu(y, a, c).reshape(b, n1, y.shape[1])

    wf, bf = params['final']
    out = _final_mm(feats[0], wf, bf[None, :], bn=512)
    return xyz, out


# speedup vs baseline: 5.6819x; 1.0930x over previous
import functools

import jax
import jax.numpy as jnp
from jax.experimental import pallas as pl

_EPS = 1e-5
_NPTS = (1024, 256, 64)
_F32 = jnp.float32


def _fps_body(start_ref, x_ref, y_ref, z_ref, fx_ref, fy_ref, fz_ref,
              fi_ref, *, npoint):
    x = x_ref[...]
    y = y_ref[...]
    z = z_ref[...]
    b, n = x.shape
    iota = jax.lax.broadcasted_iota(jnp.int32, (b, n), 1).astype(_F32)
    iop = jax.lax.broadcasted_iota(jnp.int32, (b, npoint), 1)
    far0 = start_ref[...].astype(_F32)

    def body(i, st):
        dist, far, fx, fy, fz, fi = st
        onehot = iota == far
        cx = jnp.sum(jnp.where(onehot, x, 0.0), axis=1, keepdims=True)
        cy = jnp.sum(jnp.where(onehot, y, 0.0), axis=1, keepdims=True)
        cz = jnp.sum(jnp.where(onehot, z, 0.0), axis=1, keepdims=True)
        sel = iop == i
        fx = jnp.where(sel, cx, fx)
        fy = jnp.where(sel, cy, fy)
        fz = jnp.where(sel, cz, fz)
        fi = jnp.where(sel, far.astype(jnp.int32), fi)
        d = (x - cx) ** 2 + (y - cy) ** 2 + (z - cz) ** 2
        dist = jnp.minimum(dist, d)
        m = jnp.max(dist, axis=1, keepdims=True)
        far = jnp.min(jnp.where(dist == m, iota, float(n)), axis=1,
                      keepdims=True)
        return dist, far, fx, fy, fz, fi

    init = (jnp.full((b, n), 1e10, _F32), far0,
            jnp.zeros((b, npoint), _F32), jnp.zeros((b, npoint), _F32),
            jnp.zeros((b, npoint), _F32), jnp.zeros((b, npoint), jnp.int32))
    _, _, fx, fy, fz, fi = jax.lax.fori_loop(0, npoint, body, init)
    fx_ref[...] = fx
    fy_ref[...] = fy
    fz_ref[...] = fz
    fi_ref[...] = fi


def _fps(x, y, z, start, npoint):
    b, n = x.shape
    sd = jax.ShapeDtypeStruct
    return pl.pallas_call(
        functools.partial(_fps_body, npoint=npoint),
        out_shape=[sd((b, npoint), _F32), sd((b, npoint), _F32),
                   sd((b, npoint), _F32), sd((b, npoint), jnp.int32)],
    )(start, x, y, z)


def _sqdist(a, pts3, xp_ref, yp_ref, zp_ref):
    bx, by, bz = xp_ref[0], yp_ref[0], zp_ref[0]
    sa = jnp.sum(a * a, axis=1, keepdims=True)
    sb = bx * bx + by * by + bz * bz
    ab = jax.lax.dot_general(a, pts3, (((1,), (1,)), ((), ())),
                             preferred_element_type=_F32)
    return sa + sb - 2.0 * ab


def _ball_body(fxyz_ref, xp_ref, yp_ref, zp_ref, pts_ref, out_ref, *, k, r2):
    a = fxyz_ref[0]
    d = _sqdist(a, pts_ref[0], xp_ref, yp_ref, zp_ref)
    sn, n = d.shape
    mask = d <= r2
    mf = mask.astype(_F32)
    c = mf
    s = 1
    while s < n:
        c = c + jnp.concatenate(
            [jnp.zeros((sn, s), _F32), c[:, :n - s]], axis=1)
        s *= 2
    rank = c - mf
    cnt = jnp.sum(mf, axis=1, keepdims=True)
    slot = jnp.where(mask, rank, -1.0)
    pts = pts_ref[0]
    last = jnp.broadcast_to(pts[n - 1:n, :], (sn, 3))

    def body(j, g0):
        jf = j.astype(_F32)
        oh = (slot == jf).astype(_F32)
        g = jnp.dot(oh, pts, precision=jax.lax.Precision.HIGHEST,
                    preferred_element_type=_F32)
        g0 = jnp.where(j == 0, g, g0)
        g0 = jnp.where(cnt > 0, g0, last)
        gk = jnp.where(jf < cnt, g, g0)
        out_ref[0, pl.ds(j, 1), :, :] = (gk - a)[None]
        return g0

    jax.lax.fori_loop(0, k, body, jnp.zeros((sn, 3), _F32))


def _ball_group(fxyz, xp, yp, zp, pts, k, r2, sn):
    b, s, _ = fxyz.shape
    n = pts.shape[1]
    return pl.pallas_call(
        functools.partial(_ball_body, k=k, r2=r2),
        grid=(b, s // sn),
        in_specs=[
            pl.BlockSpec((1, sn, 3), lambda i, j: (i, j, 0)),
            pl.BlockSpec((1, 1, n), lambda i, j: (i, 0, 0)),
            pl.BlockSpec((1, 1, n), lambda i, j: (i, 0, 0)),
            pl.BlockSpec((1, 1, n), lambda i, j: (i, 0, 0)),
            pl.BlockSpec((1, n, 3), lambda i, j: (i, 0, 0)),
        ],
        out_specs=pl.BlockSpec((1, k, sn, 3), lambda i, j: (i, 0, j, 0)),
        out_shape=jax.ShapeDtypeStruct((b, k, s, 3), _F32),
    )(fxyz, xp, yp, zp, pts)


def _knn_body(fxyz_ref, xp_ref, yp_ref, zp_ref, pts_ref, out_ref, *, k):
    a = fxyz_ref[0]
    pts = pts_ref[0]
    d = _sqdist(a, pts[:, 0:3], xp_ref, yp_ref, zp_ref)
    s, n = d.shape
    iota = jax.lax.broadcasted_iota(jnp.int32, (s, n), 1).astype(_F32)

    def body(j, d):
        m = jnp.min(d, axis=1, keepdims=True)
        jstar = jnp.min(jnp.where(d == m, iota, float(n)), axis=1,
                        keepdims=True)
        oh = iota == jstar
        g = jnp.dot(oh.astype(_F32), pts, precision=jax.lax.Precision.HIGHEST,
                    preferred_element_type=_F32)
        row = jnp.concatenate([g[:, :3] - a, g[:, 3:]], axis=1)
        out_ref[0, pl.ds(j, 1), :, :] = row[None]
        return jnp.where(oh, 1e30, d)

    jax.lax.fori_loop(0, k, body, d)


def _knn_group(fxyz, xp, yp, zp, pts, k):
    b, s, _ = fxyz.shape
    n, c = pts.shape[1], pts.shape[2]
    return pl.pallas_call(
        functools.partial(_knn_body, k=k),
        grid=(b,),
        in_specs=[
            pl.BlockSpec((1, s, 3), lambda i: (i, 0, 0)),
            pl.BlockSpec((1, 1, n), lambda i: (i, 0, 0)),
            pl.BlockSpec((1, 1, n), lambda i: (i, 0, 0)),
            pl.BlockSpec((1, 1, n), lambda i: (i, 0, 0)),
            pl.BlockSpec((1, n, c), lambda i: (i, 0, 0)),
        ],
        out_specs=pl.BlockSpec((1, k, s, c), lambda i: (i, 0, 0, 0)),
        out_shape=jax.ShapeDtypeStruct((b, k, s, c), _F32),
    )(fxyz, xp, yp, zp, pts)


def _mm_body(x_ref, a_ref, c_ref, w_ref, b_ref, y_ref, s1_ref, s2_ref, *,
             act):
    x = x_ref[...]
    if act:
        x = jnp.maximum(x * a_ref[...] + c_ref[...], 0.0)
    y = jnp.dot(x, w_ref[...], preferred_element_type=_F32) + b_ref[...]
    y_ref[...] = y

    @pl.when(pl.program_id(0) == 0)
    def _():
        s1_ref[...] = jnp.zeros_like(s1_ref)
        s2_ref[...] = jnp.zeros_like(s2_ref)

    s1_ref[...] += jnp.sum(y, axis=0, keepdims=True)
    s2_ref[...] += jnp.sum(y * y, axis=0, keepdims=True)


def _mm_stats(x, a, c, w, bias, act, bm=2048):
    r, ci = x.shape
    co = w.shape[1]
    bm = min(bm, r)
    sd = jax.ShapeDtypeStruct
    return pl.pallas_call(
        functools.partial(_mm_body, act=act),
        grid=(r // bm,),
        in_specs=[
            pl.BlockSpec((bm, ci), lambda i: (i, 0)),
            pl.BlockSpec((1, ci), lambda i: (0, 0)),
            pl.BlockSpec((1, ci), lambda i: (0, 0)),
            pl.BlockSpec((ci, co), lambda i: (0, 0)),
            pl.BlockSpec((1, co), lambda i: (0, 0)),
        ],
        out_specs=[
            pl.BlockSpec((bm, co), lambda i: (i, 0)),
            pl.BlockSpec((1, co), lambda i: (0, 0)),
            pl.BlockSpec((1, co), lambda i: (0, 0)),
        ],
        out_shape=[sd((r, co), _F32), sd((1, co), _F32), sd((1, co), _F32)],
    )(x, a, c, w, bias)


def _pool_body(y_ref, a_ref, c_ref, o_ref):
    t = jnp.maximum(y_ref[0] * a_ref[...][None] + c_ref[...][None], 0.0)
    o_ref[0] = jnp.max(t, axis=0)


def _pool(y, a, c):
    b, k, s, ch = y.shape
    return pl.pallas_call(
        _pool_body,
        grid=(b,),
        in_specs=[
            pl.BlockSpec((1, k, s, ch), lambda i: (i, 0, 0, 0)),
            pl.BlockSpec((1, ch), lambda i: (0, 0)),
            pl.BlockSpec((1, ch), lambda i: (0, 0)),
        ],
        out_specs=pl.BlockSpec((1, s, ch), lambda i: (i, 0, 0)),
        out_shape=jax.ShapeDtypeStruct((b, s, ch), _F32),
    )(y, a, c)


def _bn_relu_body(y_ref, a_ref, c_ref, o_ref):
    o_ref[...] = jnp.maximum(y_ref[...] * a_ref[...] + c_ref[...], 0.0)


def _bn_relu(y, a, c, bm=2048):
    r, ch = y.shape
    bm = min(bm, r)
    return pl.pallas_call(
        _bn_relu_body,
        grid=(r // bm,),
        in_specs=[
            pl.BlockSpec((bm, ch), lambda i: (i, 0)),
            pl.BlockSpec((1, ch), lambda i: (0, 0)),
            pl.BlockSpec((1, ch), lambda i: (0, 0)),
        ],
        out_specs=pl.BlockSpec((bm, ch), lambda i: (i, 0)),
        out_shape=jax.ShapeDtypeStruct((r, ch), _F32),
    )(y, a, c)


def _interp_body(xyz1_ref, xyz2_ref, x2_ref, y2_ref, z2_ref, f2_ref, o_ref):
    a = xyz1_ref[0]
    d = _sqdist(a, xyz2_ref[0], x2_ref, y2_ref, z2_ref)
    rn, n2 = d.shape
    iota = jax.lax.broadcasted_iota(jnp.int32, (rn, n2), 1).astype(_F32)
    wm = jnp.zeros((rn, n2), _F32)
    wsum = jnp.zeros((rn, 1), _F32)
    for _ in range(3):
        m = jnp.min(d, axis=1, keepdims=True)
        jstar = jnp.min(jnp.where(d == m, iota, float(n2)), axis=1,
                        keepdims=True)
        oh = iota == jstar
        w = 1.0 / jnp.maximum(m, 1e-10)
        wm = wm + jnp.where(oh, w, 0.0)
        wsum = wsum + w
        d = jnp.where(oh, 1e30, d)
    wm = wm / wsum
    o_ref[0] = jnp.dot(wm, f2_ref[0], precision=jax.lax.Precision.HIGHEST,
                       preferred_element_type=_F32)


def _interp(xyz1, xyz2, x2, y2, z2, f2, rn):
    b, n1, _ = xyz1.shape
    n2, c2 = f2.shape[1], f2.shape[2]
    rn = min(rn, n1)
    return pl.pallas_call(
        _interp_body,
        grid=(b, n1 // rn),
        in_specs=[
            pl.BlockSpec((1, rn, 3), lambda i, j: (i, j, 0)),
            pl.BlockSpec((1, n2, 3), lambda i, j: (i, 0, 0)),
            pl.BlockSpec((1, 1, n2), lambda i, j: (i, 0, 0)),
            pl.BlockSpec((1, 1, n2), lambda i, j: (i, 0, 0)),
            pl.BlockSpec((1, 1, n2), lambda i, j: (i, 0, 0)),
            pl.BlockSpec((1, n2, c2), lambda i, j: (i, 0, 0)),
        ],
        out_specs=pl.BlockSpec((1, rn, c2), lambda i, j: (i, j, 0)),
        out_shape=jax.ShapeDtypeStruct((b, n1, c2), _F32),
    )(xyz1, xyz2, x2, y2, z2, f2)


def _final_body(x_ref, w_ref, b_ref, o_ref):
    y = jnp.dot(x_ref[0], w_ref[...], preferred_element_type=_F32)
    o_ref[0] = (y + b_ref[...]).T


def _final_mm(x, w, bias, bn=512):
    b, n, ci = x.shape
    co = w.shape[1]
    return pl.pallas_call(
        _final_body,
        grid=(b, n // bn),
        in_specs=[
            pl.BlockSpec((1, bn, ci), lambda i, j: (i, j, 0)),
            pl.BlockSpec((ci, co), lambda i, j: (0, 0)),
            pl.BlockSpec((1, co), lambda i, j: (0, 0)),
        ],
        out_specs=pl.BlockSpec((1, co, bn), lambda i, j: (i, 0, j)),
        out_shape=jax.ShapeDtypeStruct((b, co, n), _F32),
    )(x, w, bias)


def _bn_affine(s1, s2, r, gamma, beta):
    mean = s1 / r
    var = s2 / r - mean * mean
    a = gamma[None, :] * jax.lax.rsqrt(var + _EPS)
    c = beta[None, :] - mean * a
    return a, c


def _mlp_rows(x, layers, last_act):
    r = x.shape[0]
    a = jnp.zeros((1, x.shape[1]), _F32)
    c = jnp.zeros((1, x.shape[1]), _F32)
    act = False
    for (w, bias, gamma, beta) in layers:
        x, s1, s2 = _mm_stats(x, a, c, w, bias[None, :], act)
        a, c = _bn_affine(s1, s2, float(r), gamma, beta)
        act = True
    return x, a, c


def kernel(pointcloud, params, numpoints):
    b, n, _ = pointcloud.shape
    xyz = pointcloud[..., 0:3]
    xp = [xyz[:, :, 0]]
    yp = [xyz[:, :, 1]]
    zp = [xyz[:, :, 2]]
    xyz3 = [xyz]
    feats = [None]

    sa_cfg = [(0.3, 32), (None, 48), (None, 48)]
    for lvl in range(3):
        radius, k = sa_cfg[lvl]
        npoint = _NPTS[lvl]
        start = (jnp.asarray(numpoints[lvl]).astype(jnp.int32)
                 - jnp.int32(npoint))
        start_arr = jnp.full((b, 1), 1, jnp.int32) * start
        fx, fy, fz, _ = _fps(xp[lvl], yp[lvl], zp[lvl], start_arr, npoint)
        fxyz = jnp.stack([fx, fy, fz], axis=-1)
        if radius is not None:
            pts = xyz3[lvl]
            grouped = _ball_group(fxyz, xp[lvl][:, None, :],
                                  yp[lvl][:, None, :], zp[lvl][:, None, :],
                                  pts, k, radius * radius, sn=512)
        else:
            pts = jnp.concatenate([xyz3[lvl], feats[lvl]], axis=-1)
            grouped = _knn_group(fxyz, xp[lvl][:, None, :],
                                 yp[lvl][:, None, :], zp[lvl][:, None, :],
                                 pts, k)
        cin = grouped.shape[-1]
        rows = grouped.reshape(b * k * npoint, cin)
        y, a, c = _mlp_rows(rows, params['sa'][lvl], last_act=False)
        co = y.shape[1]
        feat = _pool(y.reshape(b, k, npoint, co), a, c)
        xp.append(fx)
        yp.append(fy)
        zp.append(fz)
        xyz3.append(fxyz)
        feats.append(feat)

    for lvl in (2, 1, 0):
        n1 = xyz3[lvl].shape[1]
        interp = _interp(xyz3[lvl], xyz3[lvl + 1], xp[lvl + 1][:, None, :],
                         yp[lvl + 1][:, None, :], zp[lvl + 1][:, None, :],
                         feats[lvl + 1], rn=1024)
        if feats[lvl] is not None:
            x0 = jnp.concatenate([interp, feats[lvl]], axis=-1)
        else:
            x0 = interp
        rows = x0.reshape(b * n1, x0.shape[-1])
        y, a, c = _mlp_rows(rows, params['fp'][lvl], last_act=False)
        feats[lvl] = _bn_relu(y, a, c).reshape(b, n1, y.shape[1])

    wf, bf = params['final']
    out = _final_mm(feats[0], wf, bf[None, :], bn=512)
    return xyz, out
